# baseline (speedup 1.0000x reference)
import functools

import jax
import jax.numpy as jnp
from jax import lax
from jax.experimental import pallas as pl
from jax.experimental.pallas import tpu as pltpu
from jax.experimental.pallas import tpu_sc as plsc


def _sc_add_kernel(total, pe_total, ch):
    info = plsc.get_sparse_core_info()
    nw = info.num_cores * info.num_subcores
    span = total // nw
    chunks = span // ch
    nbuf = 2
    unroll = 16

    mesh = plsc.VectorSubcoreMesh(core_axis_name="c", subcore_axis_name="s")

    @functools.partial(
        pl.kernel,
        mesh=mesh,
        out_type=jax.ShapeDtypeStruct((total,), jnp.float32),
        scratch_types=[
            pltpu.VMEM((nbuf, ch), jnp.float32),
            pltpu.VMEM((nbuf, ch), jnp.float32),
            pltpu.VMEM((nbuf, ch), jnp.float32),
            pltpu.SemaphoreType.DMA,
            pltpu.SemaphoreType.DMA,
            pltpu.SemaphoreType.DMA,
            pltpu.SemaphoreType.DMA,
            pltpu.SemaphoreType.DMA,
            pltpu.SemaphoreType.DMA,
        ],
    )
    def k(x_hbm, pe_hbm, out_hbm, xv, pv, ov, sx0, sx1, sp0, sp1, so0, so1):
        sx = (sx0, sx1)
        sp = (sp0, sp1)
        so = (so0, so1)
        wid = lax.axis_index("s") * info.num_cores + lax.axis_index("c")
        base = wid * span
        pe_base = lax.rem(base, pe_total)

        def start_loads(cc, b):
            off = cc * ch
            pltpu.make_async_copy(
                x_hbm.at[pl.ds(base + off, ch)], xv.at[b], sx[b]).start()
            pltpu.make_async_copy(
                pe_hbm.at[pl.ds(pe_base + off, ch)], pv.at[b], sp[b]).start()

        def wait_loads(b):
            pltpu.make_async_copy(x_hbm.at[pl.ds(0, ch)], xv.at[b], sx[b]).wait()
            pltpu.make_async_copy(pe_hbm.at[pl.ds(0, ch)], pv.at[b], sp[b]).wait()

        def start_store(cc, b):
            pltpu.make_async_copy(
                ov.at[b], out_hbm.at[pl.ds(base + cc * ch, ch)], so[b]).start()

        def wait_store(b):
            pltpu.make_async_copy(ov.at[b], out_hbm.at[pl.ds(0, ch)], so[b]).wait()

        def compute(b):
            xvb, pvb, ovb = xv.at[b], pv.at[b], ov.at[b]

            def add_body(j, carry):
                o = j * (16 * unroll)
                for u in range(unroll):
                    s = pl.ds(o + u * 16, 16)
                    ovb[s] = xvb[s] + pvb[s]
                return carry

            lax.fori_loop(0, ch // (16 * unroll), add_body, 0)

        for b in range(nbuf):
            start_loads(b, b)
        for b in range(nbuf):
            wait_loads(b)
            compute(b)
            start_store(b, b)
            start_loads(b + nbuf, b)

        def body(kk, carry):
            for b in range(nbuf):
                cc = kk * nbuf + b
                wait_loads(b)
                wait_store(b)
                compute(b)
                start_store(cc, b)

                @pl.when(cc + nbuf < chunks)
                def _():
                    start_loads(cc + nbuf, b)

            return carry

        lax.fori_loop(1, chunks // nbuf, body, 0)

        for b in range(nbuf):
            wait_store(b)

    return k


def kernel(x, pe):
    B, S, D = x.shape
    total = B * S * D
    pe_total = S * D
    out_flat = _sc_add_kernel(total, pe_total, 16384)(
        x.reshape(total), pe.reshape(pe_total))
    return out_flat.reshape(B, S, D)


# speedup vs baseline: 5.1354x; 5.1354x over previous
import jax
import jax.numpy as jnp
from jax.experimental import pallas as pl


def _add_pe_block(x_ref, pe_ref, o_ref):
    o_ref[...] = x_ref[...] + pe_ref[...][None, :, :]


def kernel(x, pe):
    B, S, D = x.shape
    BS = 512
    return pl.pallas_call(
        _add_pe_block,
        grid=(S // BS,),
        in_specs=[
            pl.BlockSpec((B, BS, D), lambda s: (0, s, 0)),
            pl.BlockSpec((BS, D), lambda s: (s, 0)),
        ],
        out_specs=pl.BlockSpec((B, BS, D), lambda s: (0, s, 0)),
        out_shape=jax.ShapeDtypeStruct((B, S, D), x.dtype),
    )(x, pe)
